# baseline (speedup 1.0000x reference)
import functools

import jax
import jax.numpy as jnp
from jax import lax
from jax.experimental import pallas as pl
from jax.experimental.pallas import tpu as pltpu
from jax.experimental.pallas import tpu_sc as plsc

_N_NODES = 100000
_BATCH = 16384
_MEM_DIM = 128
_LANES = 16

_NC = 2
_NS = 16
_NW = _NC * _NS

_PER_TILE = _BATCH // _NS
_CHUNK = 128
_NCHUNK = _PER_TILE // _CHUNK
_VPC = _CHUNK // _LANES
_ROWS_PER_W = _BATCH // _NW
_RCHUNKS = _ROWS_PER_W // _CHUNK

_TBL_WORDS = _N_NODES + _NS * _CHUNK

_MESH = plsc.VectorSubcoreMesh(core_axis_name="c", subcore_axis_name="s")


@functools.partial(
    pl.kernel,
    out_type=jax.ShapeDtypeStruct((_BATCH, _MEM_DIM), jnp.float32),
    mesh=_MESH,
    scratch_types=[
        pltpu.VMEM_SHARED((_TBL_WORDS,), jnp.int32),
        pltpu.VMEM_SHARED((_NS * _LANES,), jnp.int32),
        pltpu.VMEM((_NCHUNK, _CHUNK), jnp.int32),
        pltpu.VMEM((_PER_TILE,), jnp.int32),
        pltpu.VMEM((_NCHUNK, _CHUNK), jnp.int32),
        pltpu.VMEM((_PER_TILE,), jnp.int32),
        pltpu.VMEM((_LANES,), jnp.int32),
        pltpu.VMEM((_NS * _LANES,), jnp.int32),
        pltpu.VMEM((_ROWS_PER_W, _MEM_DIM), jnp.float32),
        pltpu.SemaphoreType.DMA,
        pltpu.SemaphoreType.DMA,
    ],
    compiler_params=pltpu.CompilerParams(needs_layout_passes=False),
)
def _FUSED_KERNEL(idx_hbm, values_hbm, out_hbm,
                  tbl_sh, cnt_sh, idx_v, j_v, sidx_v, w_v, cntrow_v,
                  cntall_v, rows_v, sem, wsem):
    cid = lax.axis_index("c")
    sid = lax.axis_index("s")
    wid = sid * _NC + cid
    iota = lax.iota(jnp.int32, _LANES)
    dump_base = _N_NODES + sid * _CHUNK
    my_base = sid * _PER_TILE

    cp0 = pltpu.async_copy(idx_hbm.at[sid], idx_v, sem)
    cp1 = pltpu.async_copy(idx_hbm.at[sid], sidx_v, wsem)

    def _fill(v, carry):
        j_v[pl.ds(v * _LANES, _LANES)] = my_base + v * _LANES + iota
        return carry

    lax.fori_loop(0, _PER_TILE // _LANES, _fill, jnp.int32(0))
    cp0.wait()
    cp1.wait()

    def _round(_):
        cps = [
            pltpu.async_copy(
                j_v.at[pl.ds(ch * _CHUNK, _CHUNK)],
                tbl_sh.at[sidx_v.at[ch]],
                sem,
            )
            for ch in range(_NCHUNK)
        ]
        for cp in cps:
            cp.wait()
        plsc.subcore_barrier()

        cps = [
            pltpu.async_copy(
                tbl_sh.at[idx_v.at[ch]],
                w_v.at[pl.ds(ch * _CHUNK, _CHUNK)],
                sem,
            )
            for ch in range(_NCHUNK)
        ]
        for cp in cps:
            cp.wait()

        acc = jnp.zeros((_LANES,), jnp.int32)
        for ch in range(_NCHUNK):
            def _cmp(v, a, ch=ch):
                o = ch * _CHUNK + v * _LANES
                n = idx_v[ch, pl.ds(v * _LANES, _LANES)]
                t = w_v[pl.ds(o, _LANES)]
                j = my_base + o + iota
                m = j > t
                pad = dump_base + v * _LANES + iota
                sidx_v[ch, pl.ds(v * _LANES, _LANES)] = jnp.where(m, n, pad)
                return a + m.astype(jnp.int32)

            acc = lax.fori_loop(0, _VPC, _cmp, acc)

        cntrow_v[...] = jnp.broadcast_to(jnp.sum(acc), (_LANES,))
        pltpu.sync_copy(cntrow_v, cnt_sh.at[pl.ds(sid * _LANES, _LANES)])
        plsc.subcore_barrier()
        pltpu.sync_copy(cnt_sh, cntall_v)

        def _tot(r, a):
            return a + cntall_v[pl.ds(r * _LANES, _LANES)]

        tot = lax.fori_loop(0, _NS, _tot, jnp.zeros((_LANES,), jnp.int32))
        return jnp.sum(tot)

    lax.while_loop(lambda c: c > 0, _round, jnp.int32(1))

    row0 = cid * _ROWS_PER_W
    cps = [
        pltpu.async_copy(
            values_hbm.at[w_v.at[pl.ds(row0 + ch * _CHUNK, _CHUNK)]],
            rows_v.at[pl.ds(ch * _CHUNK, _CHUNK)],
            sem,
        )
        for ch in range(_RCHUNKS)
    ]
    outs = []
    for ch in range(_RCHUNKS):
        cps[ch].wait()
        outs.append(
            pltpu.async_copy(
                rows_v.at[pl.ds(ch * _CHUNK, _CHUNK)],
                out_hbm.at[pl.ds(wid * _ROWS_PER_W + ch * _CHUNK, _CHUNK)],
                wsem,
            )
        )
    for cp in outs:
        cp.wait()


def kernel(memory, node_idxs, values):
    del memory
    idx3 = node_idxs.reshape(_NS, _NCHUNK, _CHUNK)
    return _FUSED_KERNEL(idx3, values)


# speedup vs baseline: 1.0283x; 1.0283x over previous
import functools

import jax
import jax.numpy as jnp
from jax import lax
from jax.experimental import pallas as pl
from jax.experimental.pallas import tpu as pltpu
from jax.experimental.pallas import tpu_sc as plsc

_N_NODES = 100000
_BATCH = 16384
_MEM_DIM = 128
_LANES = 16

_NC = 2
_NS = 16
_NW = _NC * _NS

_PER_TILE = _BATCH // _NS
_CHUNK = 128
_NCHUNK = _PER_TILE // _CHUNK
_VPC = _CHUNK // _LANES
_ROWS_PER_W = _BATCH // _NW
_RCHUNKS = _ROWS_PER_W // _CHUNK
_RVEC = _ROWS_PER_W // _LANES
_FIXCAP = _RCHUNKS * _CHUNK + 2 * _LANES

_TBL_WORDS = _N_NODES + _NS * _CHUNK

_MESH = plsc.VectorSubcoreMesh(core_axis_name="c", subcore_axis_name="s")


@functools.partial(
    pl.kernel,
    out_type=jax.ShapeDtypeStruct((_BATCH, _MEM_DIM), jnp.float32),
    mesh=_MESH,
    scratch_types=[
        pltpu.VMEM_SHARED((_TBL_WORDS,), jnp.int32),
        pltpu.VMEM_SHARED((_NS * _LANES,), jnp.int32),
        pltpu.VMEM((_NCHUNK, _CHUNK), jnp.int32),
        pltpu.VMEM((_PER_TILE,), jnp.int32),
        pltpu.VMEM((_NCHUNK, _CHUNK), jnp.int32),
        pltpu.VMEM((_PER_TILE,), jnp.int32),
        pltpu.VMEM((_LANES,), jnp.int32),
        pltpu.VMEM((_NS * _LANES,), jnp.int32),
        pltpu.VMEM((_ROWS_PER_W, _MEM_DIM), jnp.float32),
        pltpu.VMEM((_FIXCAP,), jnp.int32),
        pltpu.VMEM((_FIXCAP,), jnp.int32),
        pltpu.VMEM((_RCHUNKS, _CHUNK), jnp.int32),
        pltpu.VMEM((_RCHUNKS, _CHUNK), jnp.int32),
        pltpu.VMEM((_CHUNK, _MEM_DIM), jnp.float32),
        pltpu.SemaphoreType.DMA,
        pltpu.SemaphoreType.DMA,
    ],
    compiler_params=pltpu.CompilerParams(needs_layout_passes=False),
)
def _FUSED_KERNEL(idx_hbm, values_hbm, out_hbm,
                  tbl_sh, cnt_sh, idx_v, j_v, sidx_v, w_v, cntrow_v,
                  cntall_v, rows_v, fixi_v, fixw_v, fixi2_v, fixw2_v,
                  frows_v, sem, wsem):
    cid = lax.axis_index("c")
    sid = lax.axis_index("s")
    wid = sid * _NC + cid
    iota = lax.iota(jnp.int32, _LANES)
    dump_base = _N_NODES + sid * _CHUNK
    my_base = sid * _PER_TILE
    out_base = wid * _ROWS_PER_W

    spec = [
        pltpu.async_copy(
            values_hbm.at[pl.ds(out_base + ch * _CHUNK, _CHUNK)],
            rows_v.at[pl.ds(ch * _CHUNK, _CHUNK)],
            wsem,
        )
        for ch in range(_RCHUNKS)
    ]

    cp0 = pltpu.async_copy(idx_hbm.at[sid], idx_v, sem)
    cp1 = pltpu.async_copy(idx_hbm.at[sid], sidx_v, sem)

    def _fill(v, carry):
        j_v[pl.ds(v * _LANES, _LANES)] = my_base + v * _LANES + iota
        return carry

    lax.fori_loop(0, _PER_TILE // _LANES, _fill, jnp.int32(0))
    cp0.wait()
    cp1.wait()

    def _round(_):
        cps = [
            pltpu.async_copy(
                j_v.at[pl.ds(ch * _CHUNK, _CHUNK)],
                tbl_sh.at[sidx_v.at[ch]],
                sem,
            )
            for ch in range(_NCHUNK)
        ]
        for cp in cps:
            cp.wait()
        plsc.subcore_barrier()

        cps = [
            pltpu.async_copy(
                tbl_sh.at[idx_v.at[ch]],
                w_v.at[pl.ds(ch * _CHUNK, _CHUNK)],
                sem,
            )
            for ch in range(_NCHUNK)
        ]
        acc = jnp.zeros((_LANES,), jnp.int32)
        for ch in range(_NCHUNK):
            cps[ch].wait()

            def _cmp(v, a, ch=ch):
                o = ch * _CHUNK + v * _LANES
                n = idx_v[ch, pl.ds(v * _LANES, _LANES)]
                t = w_v[pl.ds(o, _LANES)]
                j = my_base + o + iota
                m = j > t
                pad = dump_base + v * _LANES + iota
                sidx_v[ch, pl.ds(v * _LANES, _LANES)] = jnp.where(m, n, pad)
                return a + m.astype(jnp.int32)

            acc = lax.fori_loop(0, _VPC, _cmp, acc)

        cntrow_v[...] = jnp.broadcast_to(jnp.sum(acc), (_LANES,))
        pltpu.sync_copy(cntrow_v, cnt_sh.at[pl.ds(sid * _LANES, _LANES)])
        plsc.subcore_barrier()
        pltpu.sync_copy(cnt_sh, cntall_v)
        tot = jnp.zeros((_LANES,), jnp.int32)
        for r in range(_NS):
            tot = tot + cntall_v[pl.ds(r * _LANES, _LANES)]
        return jnp.sum(tot)

    lax.while_loop(lambda c: c > 0, _round, jnp.int32(1))
    row0 = cid * _ROWS_PER_W

    outs = []
    for ch in range(_RCHUNKS):
        spec[ch].wait()
        outs.append(
            pltpu.async_copy(
                rows_v.at[pl.ds(ch * _CHUNK, _CHUNK)],
                out_hbm.at[pl.ds(out_base + ch * _CHUNK, _CHUNK)],
                wsem,
            )
        )

    def _pre(k, carry):
        r = k * _LANES - (k // _RVEC) * _ROWS_PER_W
        fixi_v[pl.ds(k * _LANES, _LANES)] = out_base + r + iota
        fixw_v[pl.ds(k * _LANES, _LANES)] = w_v[pl.ds(row0 + r, _LANES)]
        return carry

    lax.fori_loop(0, _FIXCAP // _LANES, _pre, jnp.int32(0))

    def _compact(v, off):
        o = row0 + v * _LANES
        i_vec = out_base + v * _LANES + iota
        wv = w_v[pl.ds(o, _LANES)]
        m = wv != i_vec
        plsc.store_compressed(fixi_v.at[pl.ds(off, _LANES)], i_vec, mask=m)
        plsc.store_compressed(fixw_v.at[pl.ds(off, _LANES)], wv, mask=m)
        cnt = plsc.all_reduce_population_count(m)
        return off + jnp.max(cnt)

    n_fix = lax.fori_loop(0, _RVEC, _compact, jnp.int32(0))

    def _mk2(v, carry):
        c, r = v // _VPC, v - (v // _VPC) * _VPC
        fixi2_v[c, pl.ds(r * _LANES, _LANES)] = fixi_v[pl.ds(v * _LANES, _LANES)]
        fixw2_v[c, pl.ds(r * _LANES, _LANES)] = fixw_v[pl.ds(v * _LANES, _LANES)]
        return carry

    for v in range(_RCHUNKS * _VPC):
        _mk2(v, 0)

    for cp in outs:
        cp.wait()

    for c in range(_RCHUNKS):
        @pl.when(n_fix > c * _CHUNK)
        def _(c=c):
            pltpu.async_copy(
                values_hbm.at[fixw2_v.at[c]], frows_v, sem
            ).wait()
            pltpu.async_copy(
                frows_v, out_hbm.at[fixi2_v.at[c]], sem
            ).wait()


def kernel(memory, node_idxs, values):
    del memory
    idx3 = node_idxs.reshape(_NS, _NCHUNK, _CHUNK)
    return _FUSED_KERNEL(idx3, values)
